# Initial kernel scaffold; baseline (speedup 1.0000x reference)
import jax
import jax.numpy as jnp
from jax.experimental import pallas as pl


def kernel(point_bxyz, point_feat, W1, b1, W2, b2):
    raise NotImplementedError("write your pallas kernel here")


# speedup vs baseline: 16.6032x; 16.6032x over previous
import jax
import jax.numpy as jnp
from jax import lax
from jax.experimental import pallas as pl
from jax.experimental.pallas import tpu as pltpu
from jax.experimental.pallas import tpu_sc as plsc

N = 10000
B = 4
NB = N // B
C = 256
K = 32
STRIDE = 4
M = N // STRIDE
MB = M // B
MBP = 640
MP = B * MBP
NBP = 2560
QBLK = 128
NQB = MBP // QBLK

BIGF = 1e9
BIGI = 1 << 30

_NW = 32
_QPW = MP // _NW


def _z_body(f_ref, x_ref, wa_ref, wb_ref, o_ref):
    o_ref[...] = (
        jnp.dot(f_ref[...], wa_ref[...], preferred_element_type=jnp.float32)
        + jnp.dot(x_ref[...], wb_ref[...], preferred_element_type=jnp.float32)
    )


def _compute_z(point_feat, xyz8, w1a, w1b8):
    blk = 1000
    return pl.pallas_call(
        _z_body,
        grid=(N // blk,),
        in_specs=[
            pl.BlockSpec((blk, C), lambda i: (i, 0)),
            pl.BlockSpec((blk, 8), lambda i: (i, 0)),
            pl.BlockSpec((C, C), lambda i: (0, 0)),
            pl.BlockSpec((8, C), lambda i: (0, 0)),
        ],
        out_specs=pl.BlockSpec((blk, C), lambda i: (i, 0)),
        out_shape=jax.ShapeDtypeStruct((N, C), jnp.float32),
    )(point_feat, xyz8, w1a, w1b8)


def _knn_body(q_ref, c_ref, o_ref):
    b = pl.program_id(0)
    qx = q_ref[:, 0:1]
    qy = q_ref[:, 1:2]
    qz = q_ref[:, 2:3]
    cx = c_ref[0, 0:1, :]
    cy = c_ref[0, 1:2, :]
    cz = c_ref[0, 2:3, :]
    d = (qx - cx) ** 2 + (qy - cy) ** 2 + (qz - cz) ** 2
    ii = lax.broadcasted_iota(jnp.int32, (QBLK, NBP), 1)
    cols = []
    for _ in range(K):
        v = jnp.min(d, axis=1, keepdims=True)
        arg = jnp.min(jnp.where(d == v, ii, BIGI), axis=1, keepdims=True)
        cols.append(arg)
        d = jnp.where(ii == arg, BIGF, d)
    o_ref[...] = jnp.concatenate(cols, axis=1) + b * NB


def _knn(q_pad, c_pad):
    return pl.pallas_call(
        _knn_body,
        grid=(B, NQB),
        in_specs=[
            pl.BlockSpec((QBLK, 3), lambda b, j: (b * NQB + j, 0)),
            pl.BlockSpec((1, 3, NBP), lambda b, j: (b, 0, 0)),
        ],
        out_specs=pl.BlockSpec((QBLK, K), lambda b, j: (b * NQB + j, 0)),
        out_shape=jax.ShapeDtypeStruct((MP, K), jnp.int32),
    )(q_pad, c_pad)


def _scmax_body(z_hbm, idxf_hbm, out_hbm, idx_v, rows_v, out_v, sem):
    wid = lax.axis_index("c") * 16 + lax.axis_index("s")
    base = wid * _QPW
    pltpu.sync_copy(idxf_hbm.at[pl.ds(base * K, _QPW * K)], idx_v)

    def qloop(q, carry):
        pltpu.async_copy(z_hbm.at[idx_v.at[pl.ds(q * K, K)]], rows_v, sem).wait()
        for cc in range(C // 16):
            sl = pl.ds(cc * 16, 16)

            def kloop(k, acc):
                return jnp.maximum(acc, rows_v[k, sl])

            out_v[q, sl] = lax.fori_loop(1, K, kloop, rows_v[0, sl])
        return carry

    lax.fori_loop(0, _QPW, qloop, 0)
    pltpu.sync_copy(out_v, out_hbm.at[pl.ds(base, _QPW)])


def _segmax_sc(z, idx_flat):
    return pl.kernel(
        _scmax_body,
        out_type=jax.ShapeDtypeStruct((MP, C), jnp.float32),
        mesh=plsc.VectorSubcoreMesh(core_axis_name="c", subcore_axis_name="s"),
        scratch_types=[
            pltpu.VMEM((_QPW * K,), jnp.int32),
            pltpu.VMEM((K, C), jnp.float32),
            pltpu.VMEM((_QPW, C), jnp.float32),
            pltpu.SemaphoreType.DMA,
        ],
    )(z, idx_flat)


def _out_body(s_ref, qx_ref, wb_ref, b1_ref, w2_ref, b2_ref, o_ref):
    qv = jnp.dot(qx_ref[...], wb_ref[...], preferred_element_type=jnp.float32)
    agg = jnp.maximum(s_ref[...] - qv + b1_ref[...], 0.0)
    o_ref[...] = jnp.maximum(
        jnp.dot(agg, w2_ref[...], preferred_element_type=jnp.float32) + b2_ref[...],
        0.0,
    )


def _finish(segmax, q_pad8, w1b8, b1, w2, b2):
    return pl.pallas_call(
        _out_body,
        grid=(B,),
        in_specs=[
            pl.BlockSpec((MBP, C), lambda i: (i, 0)),
            pl.BlockSpec((MBP, 8), lambda i: (i, 0)),
            pl.BlockSpec((8, C), lambda i: (0, 0)),
            pl.BlockSpec((1, C), lambda i: (0, 0)),
            pl.BlockSpec((C, C), lambda i: (0, 0)),
            pl.BlockSpec((1, C), lambda i: (0, 0)),
        ],
        out_specs=pl.BlockSpec((MBP, C), lambda i: (i, 0)),
        out_shape=jax.ShapeDtypeStruct((MP, C), jnp.float32),
    )(segmax, q_pad8, w1b8, b1[None, :], w2, b2[None, :])


def kernel(point_bxyz, point_feat, W1, b1, W2, b2):
    new_bxyz = point_bxyz[::STRIDE]
    xyz = point_bxyz[:, 1:4]

    xyz8 = jnp.pad(xyz, ((0, 0), (0, 5)))
    w1a = W1[:C]
    w1b8 = jnp.pad(W1[C:], ((0, 5), (0, 0)))

    new_xyz = new_bxyz[:, 1:4].reshape(B, MB, 3)
    q_pad = jnp.pad(new_xyz, ((0, 0), (0, MBP - MB), (0, 0))).reshape(MP, 3)
    q_pad8 = jnp.pad(q_pad, ((0, 0), (0, 5)))

    c_pad = jnp.pad(
        xyz.reshape(B, NB, 3).transpose(0, 2, 1),
        ((0, 0), (0, 0), (0, NBP - NB)),
        constant_values=1e4,
    )

    z = _compute_z(point_feat, xyz8, w1a, w1b8)
    idx = _knn(q_pad, c_pad)
    segmax = _segmax_sc(z, idx.reshape(-1))
    out_full = _finish(segmax, q_pad8, w1b8, b1, W2, b2)

    new_feat = out_full.reshape(B, MBP, C)[:, :MB].reshape(M, C)
    return (new_bxyz, new_feat)
